# baseline (speedup 1.0000x reference)
import functools

import jax
import jax.numpy as jnp
from jax import lax
from jax.experimental import pallas as pl
from jax.experimental.pallas import tpu as pltpu
from jax.experimental.pallas import tpu_sc as plsc

N = 10000
E = 320000
HID = 128
REAL_E = E + N
BLK = 48
NBLK = 216
CH = 24
MLOOP = NBLK // 6
EPT = NBLK * BLK
EPAD = 32 * EPT
ROWS_MAIN = 632
ROWS_LAST = N - 15 * ROWS_MAIN
SWIN_MAIN = 640
SWIN_LAST = N - 15 * SWIN_MAIN
RB = 1000
GRID = N // RB


def _proj_body(x_ref, w_ref, asv_ref, adv_ref, h_ref, a_ref, m_ref):
    h = jnp.dot(x_ref[...], w_ref[...], preferred_element_type=jnp.float32)
    h_ref[...] = h
    asr = jnp.sum(h * asv_ref[...], axis=1, keepdims=True)
    adr = jnp.sum(h * adv_ref[...], axis=1, keepdims=True)
    a_ref[...] = jnp.concatenate([asr, adr], axis=1)
    m_ref[...] = jnp.concatenate(
        [jnp.max(asr).reshape(1, 1, 1), jnp.max(adr).reshape(1, 1, 1)], axis=2)


def _project(x, w, att_s, att_d):
    return pl.pallas_call(
        _proj_body,
        grid=(GRID,),
        in_specs=[
            pl.BlockSpec((RB, HID), lambda i: (i, 0)),
            pl.BlockSpec((HID, HID), lambda i: (0, 0)),
            pl.BlockSpec((1, HID), lambda i: (0, 0)),
            pl.BlockSpec((1, HID), lambda i: (0, 0)),
        ],
        out_specs=[
            pl.BlockSpec((RB, HID), lambda i: (i, 0)),
            pl.BlockSpec((RB, 2), lambda i: (i, 0)),
            pl.BlockSpec((1, 1, 2), lambda i: (i, 0, 0)),
        ],
        out_shape=[
            jax.ShapeDtypeStruct((N, HID), jnp.float32),
            jax.ShapeDtypeStruct((N, 2), jnp.float32),
            jax.ShapeDtypeStruct((GRID, 1, 2), jnp.float32),
        ],
    )(x, w, att_s.reshape(1, HID), att_d.reshape(1, HID))


def _mid_body(u_ref, s_ref, b_ref, g_ref, be_ref, w_ref, asv_ref, adv_ref,
              h_ref, a_ref, m_ref):
    sfull = s_ref[0] + s_ref[1]
    t = (u_ref[0] + u_ref[1]) / (sfull + 1e-16) + b_ref[...]
    t = jnp.maximum(t, 0.0)
    t = t * g_ref[...] + be_ref[...]
    h = jnp.dot(t, w_ref[...], preferred_element_type=jnp.float32)
    h_ref[...] = h
    asr = jnp.sum(h * asv_ref[...], axis=1, keepdims=True)
    adr = jnp.sum(h * adv_ref[...], axis=1, keepdims=True)
    a_ref[...] = jnp.concatenate([asr, adr], axis=1)
    m_ref[...] = jnp.concatenate(
        [jnp.max(asr).reshape(1, 1, 1), jnp.max(adr).reshape(1, 1, 1)], axis=2)


def _mid(u, s2, b, gscaled, beta, w, att_s, att_d):
    return pl.pallas_call(
        _mid_body,
        grid=(GRID,),
        in_specs=[
            pl.BlockSpec((2, RB, HID), lambda i: (0, i, 0)),
            pl.BlockSpec((2, RB, 1), lambda i: (0, i, 0)),
            pl.BlockSpec((1, HID), lambda i: (0, 0)),
            pl.BlockSpec((1, HID), lambda i: (0, 0)),
            pl.BlockSpec((1, HID), lambda i: (0, 0)),
            pl.BlockSpec((HID, HID), lambda i: (0, 0)),
            pl.BlockSpec((1, HID), lambda i: (0, 0)),
            pl.BlockSpec((1, HID), lambda i: (0, 0)),
        ],
        out_specs=[
            pl.BlockSpec((RB, HID), lambda i: (i, 0)),
            pl.BlockSpec((RB, 2), lambda i: (i, 0)),
            pl.BlockSpec((1, 1, 2), lambda i: (i, 0, 0)),
        ],
        out_shape=[
            jax.ShapeDtypeStruct((N, HID), jnp.float32),
            jax.ShapeDtypeStruct((N, 2), jnp.float32),
            jax.ShapeDtypeStruct((GRID, 1, 2), jnp.float32),
        ],
    )(u, s2, b.reshape(1, HID), gscaled.reshape(1, HID), beta.reshape(1, HID),
      w, att_s.reshape(1, HID), att_d.reshape(1, HID))


def _final_body(u_ref, s_ref, b_ref, o_ref):
    sfull = s_ref[0] + s_ref[1]
    t = (u_ref[0] + u_ref[1]) / (sfull + 1e-16) + b_ref[...]
    o_ref[...] = jnp.maximum(t, 0.0)


def _final(u, s2, b):
    return pl.pallas_call(
        _final_body,
        grid=(GRID,),
        in_specs=[
            pl.BlockSpec((2, RB, HID), lambda i: (0, i, 0)),
            pl.BlockSpec((2, RB, 1), lambda i: (0, i, 0)),
            pl.BlockSpec((1, HID), lambda i: (0, 0)),
        ],
        out_specs=pl.BlockSpec((RB, HID), lambda i: (i, 0)),
        out_shape=jax.ShapeDtypeStruct((N, HID), jnp.float32),
    )(u, s2, b.reshape(1, HID))


def _edge_body(h_hbm, asrc_hbm, adst_hbm, cvec_hbm, src_hbm, dst_hbm,
               u_hbm, sflat_hbm,
               asrc_t, adst_t, cvec_t, rows0_t, rows1_t, rows2_t,
               sb_t, db_t, wbl0_t, wbl1_t, slin_t,
               u_sh, s_sh, semg0, semg1, semg2, sems0, sems1, sems2,
               semw0, semw1):
    c = lax.axis_index("c")
    s = lax.axis_index("s")
    g = 2 * s + c

    rows = (rows0_t, rows1_t, rows2_t)
    semg = (semg0, semg1, semg2)
    sems = (sems0, sems1, sems2)
    wbl = (wbl0_t, wbl1_t)
    semw = (semw0, semw1)

    pltpu.sync_copy(asrc_hbm, asrc_t)
    pltpu.sync_copy(adst_hbm, adst_t)
    pltpu.sync_copy(cvec_hbm, cvec_t)
    cv = cvec_t[...]

    zero16f = jnp.zeros((16,), jnp.float32)

    def _zero_rows(i, _):
        for j in range(8):
            rows0_t[i, pl.ds(j * 16, 16)] = zero16f
        return 0

    lax.fori_loop(0, BLK, _zero_rows, 0)

    def _zero_slin(i, _):
        slin_t[pl.ds(i * 16, 16)] = zero16f
        return 0

    lax.fori_loop(0, SWIN_MAIN // 16, _zero_slin, 0)

    row0 = s * ROWS_MAIN
    s0 = s * SWIN_MAIN

    def _zero_acc(rcnt, scnt):
        for p in range(rcnt // BLK):
            pltpu.sync_copy(rows0_t, u_sh.at[pl.ds(row0 + p * BLK, BLK)])
        rem = rcnt % BLK
        pltpu.sync_copy(rows0_t.at[pl.ds(0, rem)],
                        u_sh.at[pl.ds(row0 + (rcnt - rem), rem)])
        pltpu.sync_copy(slin_t.at[pl.ds(0, scnt)], s_sh.at[pl.ds(s0, scnt)])

    @pl.when(s < 15)
    def _():
        _zero_acc(ROWS_MAIN, SWIN_MAIN)

    @pl.when(s == 15)
    def _():
        _zero_acc(ROWS_LAST, SWIN_LAST)

    plsc.subcore_barrier()

    iota16 = lax.iota(jnp.int32, 16)

    def _load_idx(r0):
        pltpu.sync_copy(src_hbm.at[pl.ds(r0, CH)], sb_t)
        pltpu.sync_copy(dst_hbm.at[pl.ds(r0, CH)], db_t)

    def _start_gather(lb, r):
        pltpu.async_copy(h_hbm.at[sb_t.at[lb]], rows[r], semg[r])

    def _wait_gather(r):
        pltpu.make_async_copy(h_hbm.at[sb_t.at[0]], rows[r], semg[r]).wait()

    def _start_scat(lb, r):
        pltpu.async_copy(rows[r], u_sh.at[db_t.at[lb]], sems[r])

    def _wait_scat(r):
        pltpu.make_async_copy(rows[r], u_sh.at[db_t.at[0]], sems[r]).wait()

    def _start_wscat(lb, p):
        pltpu.async_copy(wbl[p], s_sh.at[db_t.at[lb]], semw[p])

    def _wait_wscat(p):
        pltpu.make_async_copy(wbl[p], s_sh.at[db_t.at[0]], semw[p]).wait()

    def _mbody(m, _):
        @pl.when(jnp.logical_and(m % 4 == 0, m > 0))
        def _():
            r0 = g * NBLK + (m // 4) * CH
            _load_idx(pl.multiple_of(r0, 8))
            _start_gather(0, 0)

        for j in range(6):
            jr = j % 3
            jn = (j + 1) % 3
            jw = j % 2
            lb = 6 * (m % 4) + j
            ebase = (g * NBLK + 6 * m + j) * BLK

            _wait_gather(jr)
            for i in range(BLK // 16):
                si = sb_t[lb, pl.ds(i * 16, 16)]
                di = db_t[lb, pl.ds(i * 16, 16)]
                e = (plsc.load_gather(asrc_t, [si])
                     + plsc.load_gather(adst_t, [di]))
                e = jnp.maximum(e, 0.2 * e)
                w = jnp.exp(e - cv)
                w = jnp.where(ebase + i * 16 + iota16 < REAL_E, w, 0.0)
                wbl[jw][pl.ds(i * 16, 16)] = w
            pltpu.sync_copy(wbl[jw], s_sh.at[db_t.at[lb]], add=True)

            def _scale_row(i, _, jr=jr, jw=jw):
                wv = plsc.load_gather(wbl[jw],
                                      [jnp.full((16,), i, jnp.int32)])
                for q in range(8):
                    sl = pl.ds(q * 16, 16)
                    rows[jr][i, sl] = rows[jr][i, sl] * wv
                return 0

            lax.fori_loop(0, BLK, _scale_row, 0)

            if j < 5:
                _start_gather(lb + 1, jn)
            else:
                @pl.when(m % 4 != 3)
                def _():
                    _start_gather(lb + 1, jn)
            pltpu.sync_copy(rows[jr], u_sh.at[db_t.at[lb]], add=True)
        return 0

    _load_idx(g * NBLK)
    _start_gather(0, 0)
    lax.fori_loop(0, MLOOP, _mbody, 0)

    plsc.subcore_barrier()

    def _copy_out(rcnt, scnt):
        for p in range(rcnt // BLK):
            sl = pl.ds(row0 + p * BLK, BLK)
            pltpu.sync_copy(u_sh.at[sl], rows0_t)
            pltpu.sync_copy(rows0_t, u_hbm.at[c, sl])
        rem = rcnt % BLK
        sl = pl.ds(row0 + (rcnt - rem), rem)
        pltpu.sync_copy(u_sh.at[sl], rows0_t.at[pl.ds(0, rem)])
        pltpu.sync_copy(rows0_t.at[pl.ds(0, rem)], u_hbm.at[c, sl])
        pltpu.sync_copy(s_sh.at[pl.ds(s0, scnt)], slin_t.at[pl.ds(0, scnt)])
        pltpu.sync_copy(slin_t.at[pl.ds(0, scnt)],
                        sflat_hbm.at[pl.ds(c * N + s0, scnt)])

    @pl.when(s < 15)
    def _():
        _copy_out(ROWS_MAIN, SWIN_MAIN)

    @pl.when(s == 15)
    def _():
        _copy_out(ROWS_LAST, SWIN_LAST)


@functools.cache
def _edge_kernel():
    return pl.kernel(
        _edge_body,
        out_type=[
            jax.ShapeDtypeStruct((2, N, HID), jnp.float32),
            jax.ShapeDtypeStruct((2 * N,), jnp.float32),
        ],
        mesh=plsc.VectorSubcoreMesh(core_axis_name="c", subcore_axis_name="s"),
        compiler_params=pltpu.CompilerParams(needs_layout_passes=False),
        scratch_types=[
            pltpu.VMEM((N,), jnp.float32),
            pltpu.VMEM((N,), jnp.float32),
            pltpu.VMEM((16,), jnp.float32),
            pltpu.VMEM((BLK, HID), jnp.float32),
            pltpu.VMEM((BLK, HID), jnp.float32),
            pltpu.VMEM((BLK, HID), jnp.float32),
            pltpu.VMEM((CH, BLK), jnp.int32),
            pltpu.VMEM((CH, BLK), jnp.int32),
            pltpu.VMEM((BLK,), jnp.float32),
            pltpu.VMEM((BLK,), jnp.float32),
            pltpu.VMEM((SWIN_MAIN,), jnp.float32),
            pltpu.VMEM_SHARED((N, HID), jnp.float32),
            pltpu.VMEM_SHARED((N,), jnp.float32),
        ] + [pltpu.SemaphoreType.DMA] * 8,
    )


def _gat_edges(h, a, m, src_pad, dst_pad):
    msum = jnp.max(m[:, 0, 0]) + jnp.max(m[:, 0, 1])
    cval = jnp.maximum(msum, 0.2 * msum)
    cvec = jnp.full((16,), cval, jnp.float32)
    asrc = a[:, 0].reshape(N)
    adst = a[:, 1].reshape(N)
    u, sflat = _edge_kernel()(h, asrc, adst, cvec, src_pad, dst_pad)
    return u, sflat.reshape(2, N, 1)


def kernel(x, edge_index, W1, att_src1, att_dst1, b1, bn_gamma, bn_beta,
           W2, att_src2, att_dst2, b2):
    loop = jnp.arange(N, dtype=jnp.int32)
    padz = jnp.zeros((EPAD - REAL_E,), jnp.int32)
    src_pad = jnp.concatenate([edge_index[0], loop, padz]).reshape(-1, BLK)
    dst_pad = jnp.concatenate([edge_index[1], loop, padz]).reshape(-1, BLK)

    h1, a1, m1 = _project(x, W1, att_src1, att_dst1)
    u1, s1 = _gat_edges(h1, a1, m1, src_pad, dst_pad)

    gscaled = bn_gamma * (1.0 / jnp.sqrt(1.0 + 1e-5)).astype(jnp.float32)
    h2, a2, m2 = _mid(u1, s1, b1, gscaled, bn_beta, W2, att_src2, att_dst2)
    u2, s2 = _gat_edges(h2, a2, m2, src_pad, dst_pad)

    return _final(u2, s2, b2)


# speedup vs baseline: 1.3096x; 1.3096x over previous
import functools

import jax
import jax.numpy as jnp
from jax import lax
from jax.experimental import pallas as pl
from jax.experimental.pallas import tpu as pltpu
from jax.experimental.pallas import tpu_sc as plsc

N = 10000
E = 320000
HID = 128
REAL_E = E + N
BLK = 48
NBLK = 216
CH = 24
MLOOP = NBLK // 6
EPT = NBLK * BLK
EPAD = 32 * EPT
ROWS_MAIN = 632
ROWS_LAST = N - 15 * ROWS_MAIN
SWIN_MAIN = 640
SWIN_LAST = N - 15 * SWIN_MAIN
RB = 1000
GRID = N // RB


def _proj_body(x_ref, w_ref, asv_ref, adv_ref, h_ref, a_ref, m_ref):
    h = jnp.dot(x_ref[...], w_ref[...], preferred_element_type=jnp.float32)
    h_ref[...] = h
    asr = jnp.sum(h * asv_ref[...], axis=1, keepdims=True)
    adr = jnp.sum(h * adv_ref[...], axis=1, keepdims=True)
    a_ref[...] = jnp.concatenate([asr, adr], axis=1)
    m_ref[...] = jnp.concatenate(
        [jnp.max(asr).reshape(1, 1, 1), jnp.max(adr).reshape(1, 1, 1)], axis=2)


def _project(x, w, att_s, att_d):
    return pl.pallas_call(
        _proj_body,
        grid=(GRID,),
        in_specs=[
            pl.BlockSpec((RB, HID), lambda i: (i, 0)),
            pl.BlockSpec((HID, HID), lambda i: (0, 0)),
            pl.BlockSpec((1, HID), lambda i: (0, 0)),
            pl.BlockSpec((1, HID), lambda i: (0, 0)),
        ],
        out_specs=[
            pl.BlockSpec((RB, HID), lambda i: (i, 0)),
            pl.BlockSpec((RB, 2), lambda i: (i, 0)),
            pl.BlockSpec((1, 1, 2), lambda i: (i, 0, 0)),
        ],
        out_shape=[
            jax.ShapeDtypeStruct((N, HID), jnp.float32),
            jax.ShapeDtypeStruct((N, 2), jnp.float32),
            jax.ShapeDtypeStruct((GRID, 1, 2), jnp.float32),
        ],
    )(x, w, att_s.reshape(1, HID), att_d.reshape(1, HID))


def _mid_body(u_ref, s_ref, b_ref, g_ref, be_ref, w_ref, asv_ref, adv_ref,
              h_ref, a_ref, m_ref):
    sfull = s_ref[0] + s_ref[1]
    t = (u_ref[0] + u_ref[1]) / (sfull + 1e-16) + b_ref[...]
    t = jnp.maximum(t, 0.0)
    t = t * g_ref[...] + be_ref[...]
    h = jnp.dot(t, w_ref[...], preferred_element_type=jnp.float32)
    h_ref[...] = h
    asr = jnp.sum(h * asv_ref[...], axis=1, keepdims=True)
    adr = jnp.sum(h * adv_ref[...], axis=1, keepdims=True)
    a_ref[...] = jnp.concatenate([asr, adr], axis=1)
    m_ref[...] = jnp.concatenate(
        [jnp.max(asr).reshape(1, 1, 1), jnp.max(adr).reshape(1, 1, 1)], axis=2)


def _mid(u, s2, b, gscaled, beta, w, att_s, att_d):
    return pl.pallas_call(
        _mid_body,
        grid=(GRID,),
        in_specs=[
            pl.BlockSpec((2, RB, HID), lambda i: (0, i, 0)),
            pl.BlockSpec((2, RB, 1), lambda i: (0, i, 0)),
            pl.BlockSpec((1, HID), lambda i: (0, 0)),
            pl.BlockSpec((1, HID), lambda i: (0, 0)),
            pl.BlockSpec((1, HID), lambda i: (0, 0)),
            pl.BlockSpec((HID, HID), lambda i: (0, 0)),
            pl.BlockSpec((1, HID), lambda i: (0, 0)),
            pl.BlockSpec((1, HID), lambda i: (0, 0)),
        ],
        out_specs=[
            pl.BlockSpec((RB, HID), lambda i: (i, 0)),
            pl.BlockSpec((RB, 2), lambda i: (i, 0)),
            pl.BlockSpec((1, 1, 2), lambda i: (i, 0, 0)),
        ],
        out_shape=[
            jax.ShapeDtypeStruct((N, HID), jnp.float32),
            jax.ShapeDtypeStruct((N, 2), jnp.float32),
            jax.ShapeDtypeStruct((GRID, 1, 2), jnp.float32),
        ],
    )(u, s2, b.reshape(1, HID), gscaled.reshape(1, HID), beta.reshape(1, HID),
      w, att_s.reshape(1, HID), att_d.reshape(1, HID))


def _final_body(u_ref, s_ref, b_ref, o_ref):
    sfull = s_ref[0] + s_ref[1]
    t = (u_ref[0] + u_ref[1]) / (sfull + 1e-16) + b_ref[...]
    o_ref[...] = jnp.maximum(t, 0.0)


def _final(u, s2, b):
    return pl.pallas_call(
        _final_body,
        grid=(GRID,),
        in_specs=[
            pl.BlockSpec((2, RB, HID), lambda i: (0, i, 0)),
            pl.BlockSpec((2, RB, 1), lambda i: (0, i, 0)),
            pl.BlockSpec((1, HID), lambda i: (0, 0)),
        ],
        out_specs=pl.BlockSpec((RB, HID), lambda i: (i, 0)),
        out_shape=jax.ShapeDtypeStruct((N, HID), jnp.float32),
    )(u, s2, b.reshape(1, HID))


def _edge_body(h_hbm, asrc_hbm, adst_hbm, cvec_hbm, src_hbm, dst_hbm,
               u_hbm, sflat_hbm,
               asrc_t, adst_t, cvec_t, rows0_t, rows1_t, rows2_t,
               sb_t, db_t, wbl0_t, wbl1_t, slin_t,
               u_sh, s_sh, semg0, semg1, semg2, sems0, sems1, sems2,
               semw0, semw1):
    c = lax.axis_index("c")
    s = lax.axis_index("s")
    g = 2 * s + c

    rows = (rows0_t, rows1_t, rows2_t)
    semg = (semg0, semg1, semg2)
    sems = (sems0, sems1, sems2)
    wbl = (wbl0_t, wbl1_t)
    semw = (semw0, semw1)

    pltpu.sync_copy(asrc_hbm, asrc_t)
    pltpu.sync_copy(adst_hbm, adst_t)
    pltpu.sync_copy(cvec_hbm, cvec_t)
    cv = cvec_t[...]

    zero16f = jnp.zeros((16,), jnp.float32)

    def _zero_rows(i, _):
        for j in range(8):
            rows0_t[i, pl.ds(j * 16, 16)] = zero16f
        return 0

    lax.fori_loop(0, BLK, _zero_rows, 0)

    def _zero_slin(i, _):
        slin_t[pl.ds(i * 16, 16)] = zero16f
        return 0

    lax.fori_loop(0, SWIN_MAIN // 16, _zero_slin, 0)

    row0 = s * ROWS_MAIN
    s0 = s * SWIN_MAIN

    def _zero_acc(rcnt, scnt):
        for p in range(rcnt // BLK):
            pltpu.sync_copy(rows0_t, u_sh.at[pl.ds(row0 + p * BLK, BLK)])
        rem = rcnt % BLK
        pltpu.sync_copy(rows0_t.at[pl.ds(0, rem)],
                        u_sh.at[pl.ds(row0 + (rcnt - rem), rem)])
        pltpu.sync_copy(slin_t.at[pl.ds(0, scnt)], s_sh.at[pl.ds(s0, scnt)])

    @pl.when(s < 15)
    def _():
        _zero_acc(ROWS_MAIN, SWIN_MAIN)

    @pl.when(s == 15)
    def _():
        _zero_acc(ROWS_LAST, SWIN_LAST)

    plsc.subcore_barrier()

    iota16 = lax.iota(jnp.int32, 16)

    def _load_idx(r0):
        pltpu.sync_copy(src_hbm.at[pl.ds(r0, CH)], sb_t)
        pltpu.sync_copy(dst_hbm.at[pl.ds(r0, CH)], db_t)

    def _start_gather(lb, r):
        pltpu.async_copy(h_hbm.at[sb_t.at[lb]], rows[r], semg[r])

    def _wait_gather(r):
        pltpu.make_async_copy(h_hbm.at[sb_t.at[0]], rows[r], semg[r]).wait()

    def _start_scat(lb, r):
        pltpu.async_copy(rows[r], u_sh.at[db_t.at[lb]], sems[r])

    def _wait_scat(r):
        pltpu.make_async_copy(rows[r], u_sh.at[db_t.at[0]], sems[r]).wait()

    def _start_wscat(lb, p):
        pltpu.async_copy(wbl[p], s_sh.at[db_t.at[lb]], semw[p])

    def _wait_wscat(p):
        pltpu.make_async_copy(wbl[p], s_sh.at[db_t.at[0]], semw[p]).wait()

    def _mbody(m, _):
        @pl.when(jnp.logical_and(m % 4 == 0, m > 0))
        def _():
            r0 = g * NBLK + (m // 4) * CH
            _load_idx(pl.multiple_of(r0, 8))
            _start_gather(0, 0)

        for j in range(6):
            jr = j % 3
            jn = (j + 1) % 3
            jw = j % 2
            lb = 6 * (m % 4) + j
            ebase = (g * NBLK + 6 * m + j) * BLK

            _wait_gather(jr)
            if j < 5:
                _start_gather(lb + 1, jn)
            else:
                @pl.when(m % 4 != 3)
                def _():
                    _start_gather(lb + 1, jn)
            for i in range(BLK // 16):
                si = sb_t[lb, pl.ds(i * 16, 16)]
                di = db_t[lb, pl.ds(i * 16, 16)]
                e = (plsc.load_gather(asrc_t, [si])
                     + plsc.load_gather(adst_t, [di]))
                e = jnp.maximum(e, 0.2 * e)
                w = jnp.exp(e - cv)
                w = jnp.where(ebase + i * 16 + iota16 < REAL_E, w, 0.0)
                wbl[jw][pl.ds(i * 16, 16)] = w
            pltpu.sync_copy(wbl[jw], s_sh.at[db_t.at[lb]], add=True)

            def _scale_row(i, _, jr=jr, jw=jw):
                wv = plsc.load_gather(wbl[jw],
                                      [jnp.full((16,), i, jnp.int32)])
                for q in range(8):
                    sl = pl.ds(q * 16, 16)
                    rows[jr][i, sl] = rows[jr][i, sl] * wv
                return 0

            lax.fori_loop(0, BLK, _scale_row, 0)
            pltpu.sync_copy(rows[jr], u_sh.at[db_t.at[lb]], add=True)
        return 0

    _load_idx(g * NBLK)
    _start_gather(0, 0)
    lax.fori_loop(0, MLOOP, _mbody, 0)

    plsc.subcore_barrier()

    def _copy_out(rcnt, scnt):
        for p in range(rcnt // BLK):
            sl = pl.ds(row0 + p * BLK, BLK)
            pltpu.sync_copy(u_sh.at[sl], rows0_t)
            pltpu.sync_copy(rows0_t, u_hbm.at[c, sl])
        rem = rcnt % BLK
        sl = pl.ds(row0 + (rcnt - rem), rem)
        pltpu.sync_copy(u_sh.at[sl], rows0_t.at[pl.ds(0, rem)])
        pltpu.sync_copy(rows0_t.at[pl.ds(0, rem)], u_hbm.at[c, sl])
        pltpu.sync_copy(s_sh.at[pl.ds(s0, scnt)], slin_t.at[pl.ds(0, scnt)])
        pltpu.sync_copy(slin_t.at[pl.ds(0, scnt)],
                        sflat_hbm.at[pl.ds(c * N + s0, scnt)])

    @pl.when(s < 15)
    def _():
        _copy_out(ROWS_MAIN, SWIN_MAIN)

    @pl.when(s == 15)
    def _():
        _copy_out(ROWS_LAST, SWIN_LAST)


@functools.cache
def _edge_kernel():
    return pl.kernel(
        _edge_body,
        out_type=[
            jax.ShapeDtypeStruct((2, N, HID), jnp.float32),
            jax.ShapeDtypeStruct((2 * N,), jnp.float32),
        ],
        mesh=plsc.VectorSubcoreMesh(core_axis_name="c", subcore_axis_name="s"),
        compiler_params=pltpu.CompilerParams(needs_layout_passes=False),
        scratch_types=[
            pltpu.VMEM((N,), jnp.float32),
            pltpu.VMEM((N,), jnp.float32),
            pltpu.VMEM((16,), jnp.float32),
            pltpu.VMEM((BLK, HID), jnp.float32),
            pltpu.VMEM((BLK, HID), jnp.float32),
            pltpu.VMEM((BLK, HID), jnp.float32),
            pltpu.VMEM((CH, BLK), jnp.int32),
            pltpu.VMEM((CH, BLK), jnp.int32),
            pltpu.VMEM((BLK,), jnp.float32),
            pltpu.VMEM((BLK,), jnp.float32),
            pltpu.VMEM((SWIN_MAIN,), jnp.float32),
            pltpu.VMEM_SHARED((N, HID), jnp.float32),
            pltpu.VMEM_SHARED((N,), jnp.float32),
        ] + [pltpu.SemaphoreType.DMA] * 8,
    )


def _gat_edges(h, a, m, src_pad, dst_pad):
    msum = jnp.max(m[:, 0, 0]) + jnp.max(m[:, 0, 1])
    cval = jnp.maximum(msum, 0.2 * msum)
    cvec = jnp.full((16,), cval, jnp.float32)
    asrc = a[:, 0].reshape(N)
    adst = a[:, 1].reshape(N)
    u, sflat = _edge_kernel()(h, asrc, adst, cvec, src_pad, dst_pad)
    return u, sflat.reshape(2, N, 1)


def kernel(x, edge_index, W1, att_src1, att_dst1, b1, bn_gamma, bn_beta,
           W2, att_src2, att_dst2, b2):
    loop = jnp.arange(N, dtype=jnp.int32)
    padz = jnp.zeros((EPAD - REAL_E,), jnp.int32)
    src_pad = jnp.concatenate([edge_index[0], loop, padz]).reshape(-1, BLK)
    dst_pad = jnp.concatenate([edge_index[1], loop, padz]).reshape(-1, BLK)

    h1, a1, m1 = _project(x, W1, att_src1, att_dst1)
    u1, s1 = _gat_edges(h1, a1, m1, src_pad, dst_pad)

    gscaled = bn_gamma * (1.0 / jnp.sqrt(1.0 + 1e-5)).astype(jnp.float32)
    h2, a2, m2 = _mid(u1, s1, b1, gscaled, bn_beta, W2, att_src2, att_dst2)
    u2, s2 = _gat_edges(h2, a2, m2, src_pad, dst_pad)

    return _final(u2, s2, b2)


# speedup vs baseline: 1.3289x; 1.0148x over previous
import functools

import jax
import jax.numpy as jnp
from jax import lax
from jax.experimental import pallas as pl
from jax.experimental.pallas import tpu as pltpu
from jax.experimental.pallas import tpu_sc as plsc

N = 10000
E = 320000
HID = 128
REAL_E = E + N
BLK = 48
NBLK = 216
CH = 24
MLOOP = NBLK // 6
EPT = NBLK * BLK
EPAD = 32 * EPT
ROWS_MAIN = 632
ROWS_LAST = N - 15 * ROWS_MAIN
SWIN_MAIN = 640
SWIN_LAST = N - 15 * SWIN_MAIN
RB = 1000
GRID = N // RB


def _proj_body(x_ref, w_ref, asv_ref, adv_ref, h_ref, a_ref, m_ref):
    h = jnp.dot(x_ref[...], w_ref[...], preferred_element_type=jnp.float32)
    h_ref[...] = h
    asr = jnp.sum(h * asv_ref[...], axis=1, keepdims=True)
    adr = jnp.sum(h * adv_ref[...], axis=1, keepdims=True)
    a_ref[...] = jnp.concatenate([asr, adr], axis=1)
    m_ref[...] = jnp.concatenate(
        [jnp.max(asr).reshape(1, 1, 1), jnp.max(adr).reshape(1, 1, 1)], axis=2)


def _project(x, w, att_s, att_d):
    return pl.pallas_call(
        _proj_body,
        grid=(GRID,),
        in_specs=[
            pl.BlockSpec((RB, HID), lambda i: (i, 0)),
            pl.BlockSpec((HID, HID), lambda i: (0, 0)),
            pl.BlockSpec((1, HID), lambda i: (0, 0)),
            pl.BlockSpec((1, HID), lambda i: (0, 0)),
        ],
        out_specs=[
            pl.BlockSpec((RB, HID), lambda i: (i, 0)),
            pl.BlockSpec((RB, 2), lambda i: (i, 0)),
            pl.BlockSpec((1, 1, 2), lambda i: (i, 0, 0)),
        ],
        out_shape=[
            jax.ShapeDtypeStruct((N, HID), jnp.float32),
            jax.ShapeDtypeStruct((N, 2), jnp.float32),
            jax.ShapeDtypeStruct((GRID, 1, 2), jnp.float32),
        ],
    )(x, w, att_s.reshape(1, HID), att_d.reshape(1, HID))


def _mid_body(u_ref, s_ref, b_ref, g_ref, be_ref, w_ref, asv_ref, adv_ref,
              h_ref, a_ref, m_ref):
    sfull = s_ref[0] + s_ref[1]
    t = (u_ref[0] + u_ref[1]) / (sfull + 1e-16) + b_ref[...]
    t = jnp.maximum(t, 0.0)
    t = t * g_ref[...] + be_ref[...]
    h = jnp.dot(t, w_ref[...], preferred_element_type=jnp.float32)
    h_ref[...] = h
    asr = jnp.sum(h * asv_ref[...], axis=1, keepdims=True)
    adr = jnp.sum(h * adv_ref[...], axis=1, keepdims=True)
    a_ref[...] = jnp.concatenate([asr, adr], axis=1)
    m_ref[...] = jnp.concatenate(
        [jnp.max(asr).reshape(1, 1, 1), jnp.max(adr).reshape(1, 1, 1)], axis=2)


def _mid(u, s2, b, gscaled, beta, w, att_s, att_d):
    return pl.pallas_call(
        _mid_body,
        grid=(GRID,),
        in_specs=[
            pl.BlockSpec((2, RB, HID), lambda i: (0, i, 0)),
            pl.BlockSpec((2, RB, 1), lambda i: (0, i, 0)),
            pl.BlockSpec((1, HID), lambda i: (0, 0)),
            pl.BlockSpec((1, HID), lambda i: (0, 0)),
            pl.BlockSpec((1, HID), lambda i: (0, 0)),
            pl.BlockSpec((HID, HID), lambda i: (0, 0)),
            pl.BlockSpec((1, HID), lambda i: (0, 0)),
            pl.BlockSpec((1, HID), lambda i: (0, 0)),
        ],
        out_specs=[
            pl.BlockSpec((RB, HID), lambda i: (i, 0)),
            pl.BlockSpec((RB, 2), lambda i: (i, 0)),
            pl.BlockSpec((1, 1, 2), lambda i: (i, 0, 0)),
        ],
        out_shape=[
            jax.ShapeDtypeStruct((N, HID), jnp.float32),
            jax.ShapeDtypeStruct((N, 2), jnp.float32),
            jax.ShapeDtypeStruct((GRID, 1, 2), jnp.float32),
        ],
    )(u, s2, b.reshape(1, HID), gscaled.reshape(1, HID), beta.reshape(1, HID),
      w, att_s.reshape(1, HID), att_d.reshape(1, HID))


def _final_body(u_ref, s_ref, b_ref, o_ref):
    sfull = s_ref[0] + s_ref[1]
    t = (u_ref[0] + u_ref[1]) / (sfull + 1e-16) + b_ref[...]
    o_ref[...] = jnp.maximum(t, 0.0)


def _final(u, s2, b):
    return pl.pallas_call(
        _final_body,
        grid=(GRID,),
        in_specs=[
            pl.BlockSpec((2, RB, HID), lambda i: (0, i, 0)),
            pl.BlockSpec((2, RB, 1), lambda i: (0, i, 0)),
            pl.BlockSpec((1, HID), lambda i: (0, 0)),
        ],
        out_specs=pl.BlockSpec((RB, HID), lambda i: (i, 0)),
        out_shape=jax.ShapeDtypeStruct((N, HID), jnp.float32),
    )(u, s2, b.reshape(1, HID))


def _edge_body(h_hbm, asrc_hbm, adst_hbm, cvec_hbm, src_hbm, dst_hbm,
               u_hbm, sflat_hbm,
               asrc_t, adst_t, cvec_t, rows0_t, rows1_t, rows2_t,
               sb_t, db_t, wbl0_t, wbl1_t, slin_t,
               u_sh, s_sh, semg0, semg1, semg2, sems0, sems1, sems2,
               semw0, semw1):
    c = lax.axis_index("c")
    s = lax.axis_index("s")
    g = 2 * s + c

    rows = (rows0_t, rows1_t, rows2_t)
    semg = (semg0, semg1, semg2)
    sems = (sems0, sems1, sems2)
    wbl = (wbl0_t, wbl1_t)
    semw = (semw0, semw1)

    pltpu.sync_copy(asrc_hbm, asrc_t)
    pltpu.sync_copy(adst_hbm, adst_t)
    pltpu.sync_copy(cvec_hbm, cvec_t)
    cv = cvec_t[...]

    zero16f = jnp.zeros((16,), jnp.float32)

    def _zero_rows(i, _):
        for j in range(8):
            rows0_t[i, pl.ds(j * 16, 16)] = zero16f
        return 0

    lax.fori_loop(0, BLK, _zero_rows, 0)

    def _zero_slin(i, _):
        slin_t[pl.ds(i * 16, 16)] = zero16f
        return 0

    lax.fori_loop(0, SWIN_MAIN // 16, _zero_slin, 0)

    row0 = s * ROWS_MAIN
    s0 = s * SWIN_MAIN

    def _zero_acc(rcnt, scnt):
        for p in range(rcnt // BLK):
            pltpu.sync_copy(rows0_t, u_sh.at[pl.ds(row0 + p * BLK, BLK)])
        rem = rcnt % BLK
        pltpu.sync_copy(rows0_t.at[pl.ds(0, rem)],
                        u_sh.at[pl.ds(row0 + (rcnt - rem), rem)])
        pltpu.sync_copy(slin_t.at[pl.ds(0, scnt)], s_sh.at[pl.ds(s0, scnt)])

    @pl.when(s < 15)
    def _():
        _zero_acc(ROWS_MAIN, SWIN_MAIN)

    @pl.when(s == 15)
    def _():
        _zero_acc(ROWS_LAST, SWIN_LAST)

    plsc.subcore_barrier()

    iota16 = lax.iota(jnp.int32, 16)

    def _load_idx(r0):
        pltpu.sync_copy(src_hbm.at[pl.ds(r0, CH)], sb_t)
        pltpu.sync_copy(dst_hbm.at[pl.ds(r0, CH)], db_t)

    def _start_gather(lb, r):
        pltpu.async_copy(h_hbm.at[sb_t.at[lb]], rows[r], semg[r])

    def _wait_gather(r):
        pltpu.make_async_copy(h_hbm.at[sb_t.at[0]], rows[r], semg[r]).wait()

    def _start_scat(lb, r):
        pltpu.async_copy(rows[r], u_sh.at[db_t.at[lb]], sems[r])

    def _wait_scat(r):
        pltpu.make_async_copy(rows[r], u_sh.at[db_t.at[0]], sems[r]).wait()

    def _start_wscat(lb, p):
        pltpu.async_copy(wbl[p], s_sh.at[db_t.at[lb]], semw[p])

    def _wait_wscat(p):
        pltpu.make_async_copy(wbl[p], s_sh.at[db_t.at[0]], semw[p]).wait()

    def _mbody(m, _):
        @pl.when(jnp.logical_and(m % 4 == 0, m > 0))
        def _():
            r0 = g * NBLK + (m // 4) * CH
            _load_idx(pl.multiple_of(r0, 8))
            _start_gather(0, 0)

        scat_d = [None] * 6
        wscat_d = [None] * 6
        for j in range(6):
            jr = j % 3
            jn = (j + 1) % 3
            jw = j % 2
            lb = 6 * (m % 4) + j
            ebase = (g * NBLK + 6 * m + j) * BLK

            _wait_gather(jr)
            if j >= 2:
                scat_d[j - 2].wait()
            if j < 5:
                _start_gather(lb + 1, jn)
            else:
                @pl.when(m % 4 != 3)
                def _():
                    _start_gather(lb + 1, jn)
            if j >= 2:
                wscat_d[j - 2].wait()
            for i in range(BLK // 16):
                si = sb_t[lb, pl.ds(i * 16, 16)]
                di = db_t[lb, pl.ds(i * 16, 16)]
                e = (plsc.load_gather(asrc_t, [si])
                     + plsc.load_gather(adst_t, [di]))
                e = jnp.maximum(e, 0.2 * e)
                w = jnp.exp(e - cv)
                w = jnp.where(ebase + i * 16 + iota16 < REAL_E, w, 0.0)
                wbl[jw][pl.ds(i * 16, 16)] = w
            wscat_d[j] = pltpu.async_copy(
                wbl[jw], s_sh.at[db_t.at[lb]], semw[jw], add=True)

            def _scale_row(i, _, jr=jr, jw=jw):
                wv = plsc.load_gather(wbl[jw],
                                      [jnp.full((16,), i, jnp.int32)])
                for q in range(8):
                    sl = pl.ds(q * 16, 16)
                    rows[jr][i, sl] = rows[jr][i, sl] * wv
                return 0

            lax.fori_loop(0, BLK, _scale_row, 0)
            scat_d[j] = pltpu.async_copy(
                rows[jr], u_sh.at[db_t.at[lb]], sems[jr], add=True)

        scat_d[4].wait()
        scat_d[5].wait()
        wscat_d[4].wait()
        wscat_d[5].wait()
        return 0

    _load_idx(g * NBLK)
    _start_gather(0, 0)
    lax.fori_loop(0, MLOOP, _mbody, 0)

    plsc.subcore_barrier()

    def _copy_out(rcnt, scnt):
        for p in range(rcnt // BLK):
            sl = pl.ds(row0 + p * BLK, BLK)
            pltpu.sync_copy(u_sh.at[sl], rows0_t)
            pltpu.sync_copy(rows0_t, u_hbm.at[c, sl])
        rem = rcnt % BLK
        sl = pl.ds(row0 + (rcnt - rem), rem)
        pltpu.sync_copy(u_sh.at[sl], rows0_t.at[pl.ds(0, rem)])
        pltpu.sync_copy(rows0_t.at[pl.ds(0, rem)], u_hbm.at[c, sl])
        pltpu.sync_copy(s_sh.at[pl.ds(s0, scnt)], slin_t.at[pl.ds(0, scnt)])
        pltpu.sync_copy(slin_t.at[pl.ds(0, scnt)],
                        sflat_hbm.at[pl.ds(c * N + s0, scnt)])

    @pl.when(s < 15)
    def _():
        _copy_out(ROWS_MAIN, SWIN_MAIN)

    @pl.when(s == 15)
    def _():
        _copy_out(ROWS_LAST, SWIN_LAST)


@functools.cache
def _edge_kernel():
    return pl.kernel(
        _edge_body,
        out_type=[
            jax.ShapeDtypeStruct((2, N, HID), jnp.float32),
            jax.ShapeDtypeStruct((2 * N,), jnp.float32),
        ],
        mesh=plsc.VectorSubcoreMesh(core_axis_name="c", subcore_axis_name="s"),
        compiler_params=pltpu.CompilerParams(needs_layout_passes=False),
        scratch_types=[
            pltpu.VMEM((N,), jnp.float32),
            pltpu.VMEM((N,), jnp.float32),
            pltpu.VMEM((16,), jnp.float32),
            pltpu.VMEM((BLK, HID), jnp.float32),
            pltpu.VMEM((BLK, HID), jnp.float32),
            pltpu.VMEM((BLK, HID), jnp.float32),
            pltpu.VMEM((CH, BLK), jnp.int32),
            pltpu.VMEM((CH, BLK), jnp.int32),
            pltpu.VMEM((BLK,), jnp.float32),
            pltpu.VMEM((BLK,), jnp.float32),
            pltpu.VMEM((SWIN_MAIN,), jnp.float32),
            pltpu.VMEM_SHARED((N, HID), jnp.float32),
            pltpu.VMEM_SHARED((N,), jnp.float32),
        ] + [pltpu.SemaphoreType.DMA] * 8,
    )


def _gat_edges(h, a, m, src_pad, dst_pad):
    msum = jnp.max(m[:, 0, 0]) + jnp.max(m[:, 0, 1])
    cval = jnp.maximum(msum, 0.2 * msum)
    cvec = jnp.full((16,), cval, jnp.float32)
    asrc = a[:, 0].reshape(N)
    adst = a[:, 1].reshape(N)
    u, sflat = _edge_kernel()(h, asrc, adst, cvec, src_pad, dst_pad)
    return u, sflat.reshape(2, N, 1)


def kernel(x, edge_index, W1, att_src1, att_dst1, b1, bn_gamma, bn_beta,
           W2, att_src2, att_dst2, b2):
    loop = jnp.arange(N, dtype=jnp.int32)
    padz = jnp.zeros((EPAD - REAL_E,), jnp.int32)
    src_pad = jnp.concatenate([edge_index[0], loop, padz]).reshape(-1, BLK)
    dst_pad = jnp.concatenate([edge_index[1], loop, padz]).reshape(-1, BLK)

    h1, a1, m1 = _project(x, W1, att_src1, att_dst1)
    u1, s1 = _gat_edges(h1, a1, m1, src_pad, dst_pad)

    gscaled = bn_gamma * (1.0 / jnp.sqrt(1.0 + 1e-5)).astype(jnp.float32)
    h2, a2, m2 = _mid(u1, s1, b1, gscaled, bn_beta, W2, att_src2, att_dst2)
    u2, s2 = _gat_edges(h2, a2, m2, src_pad, dst_pad)

    return _final(u2, s2, b2)


# speedup vs baseline: 1.3427x; 1.0104x over previous
import functools

import jax
import jax.numpy as jnp
from jax import lax
from jax.experimental import pallas as pl
from jax.experimental.pallas import tpu as pltpu
from jax.experimental.pallas import tpu_sc as plsc

N = 10000
E = 320000
HID = 128
REAL_E = E + N
BLK = 48
NBLK = 216
CH = 24
MLOOP = NBLK // 6
EPT = NBLK * BLK
EPAD = 32 * EPT
ROWS_MAIN = 632
ROWS_LAST = N - 15 * ROWS_MAIN
SWIN_MAIN = 640
SWIN_LAST = N - 15 * SWIN_MAIN
RB = 1000
GRID = N // RB


def _proj_body(x_ref, w_ref, asv_ref, adv_ref, h_ref, a_ref, m_ref):
    h = jnp.dot(x_ref[...], w_ref[...], preferred_element_type=jnp.float32)
    h_ref[...] = h
    asr = jnp.sum(h * asv_ref[...], axis=1, keepdims=True)
    adr = jnp.sum(h * adv_ref[...], axis=1, keepdims=True)
    a_ref[...] = jnp.concatenate([asr, adr], axis=1)
    m_ref[...] = jnp.concatenate(
        [jnp.max(asr).reshape(1, 1, 1), jnp.max(adr).reshape(1, 1, 1)], axis=2)


def _project(x, w, att_s, att_d):
    return pl.pallas_call(
        _proj_body,
        grid=(GRID,),
        in_specs=[
            pl.BlockSpec((RB, HID), lambda i: (i, 0)),
            pl.BlockSpec((HID, HID), lambda i: (0, 0)),
            pl.BlockSpec((1, HID), lambda i: (0, 0)),
            pl.BlockSpec((1, HID), lambda i: (0, 0)),
        ],
        out_specs=[
            pl.BlockSpec((RB, HID), lambda i: (i, 0)),
            pl.BlockSpec((RB, 2), lambda i: (i, 0)),
            pl.BlockSpec((1, 1, 2), lambda i: (i, 0, 0)),
        ],
        out_shape=[
            jax.ShapeDtypeStruct((N, HID), jnp.float32),
            jax.ShapeDtypeStruct((N, 2), jnp.float32),
            jax.ShapeDtypeStruct((GRID, 1, 2), jnp.float32),
        ],
    )(x, w, att_s.reshape(1, HID), att_d.reshape(1, HID))


def _mid_body(u_ref, s_ref, b_ref, g_ref, be_ref, w_ref, asv_ref, adv_ref,
              h_ref, a_ref, m_ref):
    sfull = s_ref[0] + s_ref[1]
    t = (u_ref[0] + u_ref[1]) / (sfull + 1e-16) + b_ref[...]
    t = jnp.maximum(t, 0.0)
    t = t * g_ref[...] + be_ref[...]
    h = jnp.dot(t, w_ref[...], preferred_element_type=jnp.float32)
    h_ref[...] = h
    asr = jnp.sum(h * asv_ref[...], axis=1, keepdims=True)
    adr = jnp.sum(h * adv_ref[...], axis=1, keepdims=True)
    a_ref[...] = jnp.concatenate([asr, adr], axis=1)
    m_ref[...] = jnp.concatenate(
        [jnp.max(asr).reshape(1, 1, 1), jnp.max(adr).reshape(1, 1, 1)], axis=2)


def _mid(u, s2, b, gscaled, beta, w, att_s, att_d):
    return pl.pallas_call(
        _mid_body,
        grid=(GRID,),
        in_specs=[
            pl.BlockSpec((2, RB, HID), lambda i: (0, i, 0)),
            pl.BlockSpec((2, RB, 1), lambda i: (0, i, 0)),
            pl.BlockSpec((1, HID), lambda i: (0, 0)),
            pl.BlockSpec((1, HID), lambda i: (0, 0)),
            pl.BlockSpec((1, HID), lambda i: (0, 0)),
            pl.BlockSpec((HID, HID), lambda i: (0, 0)),
            pl.BlockSpec((1, HID), lambda i: (0, 0)),
            pl.BlockSpec((1, HID), lambda i: (0, 0)),
        ],
        out_specs=[
            pl.BlockSpec((RB, HID), lambda i: (i, 0)),
            pl.BlockSpec((RB, 2), lambda i: (i, 0)),
            pl.BlockSpec((1, 1, 2), lambda i: (i, 0, 0)),
        ],
        out_shape=[
            jax.ShapeDtypeStruct((N, HID), jnp.float32),
            jax.ShapeDtypeStruct((N, 2), jnp.float32),
            jax.ShapeDtypeStruct((GRID, 1, 2), jnp.float32),
        ],
    )(u, s2, b.reshape(1, HID), gscaled.reshape(1, HID), beta.reshape(1, HID),
      w, att_s.reshape(1, HID), att_d.reshape(1, HID))


def _final_body(u_ref, s_ref, b_ref, o_ref):
    sfull = s_ref[0] + s_ref[1]
    t = (u_ref[0] + u_ref[1]) / (sfull + 1e-16) + b_ref[...]
    o_ref[...] = jnp.maximum(t, 0.0)


def _final(u, s2, b):
    return pl.pallas_call(
        _final_body,
        grid=(GRID,),
        in_specs=[
            pl.BlockSpec((2, RB, HID), lambda i: (0, i, 0)),
            pl.BlockSpec((2, RB, 1), lambda i: (0, i, 0)),
            pl.BlockSpec((1, HID), lambda i: (0, 0)),
        ],
        out_specs=pl.BlockSpec((RB, HID), lambda i: (i, 0)),
        out_shape=jax.ShapeDtypeStruct((N, HID), jnp.float32),
    )(u, s2, b.reshape(1, HID))


def _edge_body(h_hbm, asrc_hbm, adst_hbm, cvec_hbm, src_hbm, dst_hbm,
               u_hbm, sflat_hbm,
               asrc_t, adst_t, cvec_t, rows0_t, rows1_t, rows2_t,
               sb_t, db_t, wbl0_t, wbl1_t, slin_t,
               u_sh, s_sh, semg0, semg1, semg2, sems0, sems1, sems2,
               semw0, semw1):
    c = lax.axis_index("c")
    s = lax.axis_index("s")
    g = 2 * s + c

    rows = (rows0_t, rows1_t, rows2_t)
    semg = (semg0, semg1, semg2)
    sems = (sems0, sems1, sems2)
    wbl = (wbl0_t, wbl1_t)
    semw = (semw0, semw1)

    pltpu.sync_copy(asrc_hbm, asrc_t)
    pltpu.sync_copy(adst_hbm, adst_t)
    pltpu.sync_copy(cvec_hbm, cvec_t)
    cv = cvec_t[...]

    zero16f = jnp.zeros((16,), jnp.float32)

    def _zero_rows(i, _):
        for j in range(8):
            rows0_t[i, pl.ds(j * 16, 16)] = zero16f
        return 0

    lax.fori_loop(0, BLK, _zero_rows, 0)

    def _zero_slin(i, _):
        slin_t[pl.ds(i * 16, 16)] = zero16f
        return 0

    lax.fori_loop(0, SWIN_MAIN // 16, _zero_slin, 0)

    row0 = s * ROWS_MAIN
    s0 = s * SWIN_MAIN

    def _zero_acc(rcnt, scnt):
        for p in range(rcnt // BLK):
            pltpu.sync_copy(rows0_t, u_sh.at[pl.ds(row0 + p * BLK, BLK)])
        rem = rcnt % BLK
        pltpu.sync_copy(rows0_t.at[pl.ds(0, rem)],
                        u_sh.at[pl.ds(row0 + (rcnt - rem), rem)])
        pltpu.sync_copy(slin_t.at[pl.ds(0, scnt)], s_sh.at[pl.ds(s0, scnt)])

    @pl.when(s < 15)
    def _():
        _zero_acc(ROWS_MAIN, SWIN_MAIN)

    @pl.when(s == 15)
    def _():
        _zero_acc(ROWS_LAST, SWIN_LAST)

    plsc.subcore_barrier()

    iota16 = lax.iota(jnp.int32, 16)

    def _load_idx(r0):
        pltpu.sync_copy(src_hbm.at[pl.ds(r0, CH)], sb_t)
        pltpu.sync_copy(dst_hbm.at[pl.ds(r0, CH)], db_t)

    def _start_gather(lb, r):
        pltpu.async_copy(h_hbm.at[sb_t.at[lb]], rows[r], semg[r])

    def _wait_gather(r):
        pltpu.make_async_copy(h_hbm.at[sb_t.at[0]], rows[r], semg[r]).wait()

    def _start_scat(lb, r):
        pltpu.async_copy(rows[r], u_sh.at[db_t.at[lb]], sems[r])

    def _wait_scat(r):
        pltpu.make_async_copy(rows[r], u_sh.at[db_t.at[0]], sems[r]).wait()

    def _start_wscat(lb, p):
        pltpu.async_copy(wbl[p], s_sh.at[db_t.at[lb]], semw[p])

    def _wait_wscat(p):
        pltpu.make_async_copy(wbl[p], s_sh.at[db_t.at[0]], semw[p]).wait()

    def _mbody(m, _):
        @pl.when(jnp.logical_and(m % 4 == 0, m > 0))
        def _():
            r0 = g * NBLK + (m // 4) * CH
            _load_idx(pl.multiple_of(r0, 8))
            _start_gather(0, 0)

        scat_d = [None] * 6
        wscat_d = [None] * 6
        for j in range(6):
            jr = j % 3
            jn = (j + 1) % 3
            jw = j % 2
            lb = 6 * (m % 4) + j
            ebase = (g * NBLK + 6 * m + j) * BLK

            _wait_gather(jr)
            if j >= 2:
                scat_d[j - 2].wait()
            if j < 5:
                _start_gather(lb + 1, jn)
            else:
                @pl.when(m % 4 != 3)
                def _():
                    _start_gather(lb + 1, jn)
            if j >= 2:
                wscat_d[j - 2].wait()
            for i in range(BLK // 16):
                si = sb_t[lb, pl.ds(i * 16, 16)]
                di = db_t[lb, pl.ds(i * 16, 16)]
                e = (plsc.load_gather(asrc_t, [si])
                     + plsc.load_gather(adst_t, [di]))
                e = jnp.maximum(e, 0.2 * e)
                w = jnp.exp(e - cv)
                w = jnp.where(ebase + i * 16 + iota16 < REAL_E, w, 0.0)
                wbl[jw][pl.ds(i * 16, 16)] = w
            wscat_d[j] = pltpu.async_copy(
                wbl[jw], s_sh.at[db_t.at[lb]], semw[jw], add=True)

            def _scale_row(i, _, jr=jr, jw=jw):
                wv = plsc.load_gather(wbl[jw],
                                      [jnp.full((16,), i, jnp.int32)])
                for q in range(8):
                    sl = pl.ds(q * 16, 16)
                    rows[jr][i, sl] = rows[jr][i, sl] * wv
                return 0

            lax.fori_loop(0, BLK, _scale_row, 0, unroll=4)
            scat_d[j] = pltpu.async_copy(
                rows[jr], u_sh.at[db_t.at[lb]], sems[jr], add=True)

        scat_d[4].wait()
        scat_d[5].wait()
        wscat_d[4].wait()
        wscat_d[5].wait()
        return 0

    _load_idx(g * NBLK)
    _start_gather(0, 0)
    lax.fori_loop(0, MLOOP, _mbody, 0)

    plsc.subcore_barrier()

    def _copy_out(rcnt, scnt):
        for p in range(rcnt // BLK):
            sl = pl.ds(row0 + p * BLK, BLK)
            pltpu.sync_copy(u_sh.at[sl], rows0_t)
            pltpu.sync_copy(rows0_t, u_hbm.at[c, sl])
        rem = rcnt % BLK
        sl = pl.ds(row0 + (rcnt - rem), rem)
        pltpu.sync_copy(u_sh.at[sl], rows0_t.at[pl.ds(0, rem)])
        pltpu.sync_copy(rows0_t.at[pl.ds(0, rem)], u_hbm.at[c, sl])
        pltpu.sync_copy(s_sh.at[pl.ds(s0, scnt)], slin_t.at[pl.ds(0, scnt)])
        pltpu.sync_copy(slin_t.at[pl.ds(0, scnt)],
                        sflat_hbm.at[pl.ds(c * N + s0, scnt)])

    @pl.when(s < 15)
    def _():
        _copy_out(ROWS_MAIN, SWIN_MAIN)

    @pl.when(s == 15)
    def _():
        _copy_out(ROWS_LAST, SWIN_LAST)


@functools.cache
def _edge_kernel():
    return pl.kernel(
        _edge_body,
        out_type=[
            jax.ShapeDtypeStruct((2, N, HID), jnp.float32),
            jax.ShapeDtypeStruct((2 * N,), jnp.float32),
        ],
        mesh=plsc.VectorSubcoreMesh(core_axis_name="c", subcore_axis_name="s"),
        compiler_params=pltpu.CompilerParams(needs_layout_passes=False),
        scratch_types=[
            pltpu.VMEM((N,), jnp.float32),
            pltpu.VMEM((N,), jnp.float32),
            pltpu.VMEM((16,), jnp.float32),
            pltpu.VMEM((BLK, HID), jnp.float32),
            pltpu.VMEM((BLK, HID), jnp.float32),
            pltpu.VMEM((BLK, HID), jnp.float32),
            pltpu.VMEM((CH, BLK), jnp.int32),
            pltpu.VMEM((CH, BLK), jnp.int32),
            pltpu.VMEM((BLK,), jnp.float32),
            pltpu.VMEM((BLK,), jnp.float32),
            pltpu.VMEM((SWIN_MAIN,), jnp.float32),
            pltpu.VMEM_SHARED((N, HID), jnp.float32),
            pltpu.VMEM_SHARED((N,), jnp.float32),
        ] + [pltpu.SemaphoreType.DMA] * 8,
    )


def _gat_edges(h, a, m, src_pad, dst_pad):
    msum = jnp.max(m[:, 0, 0]) + jnp.max(m[:, 0, 1])
    cval = jnp.maximum(msum, 0.2 * msum)
    cvec = jnp.full((16,), cval, jnp.float32)
    asrc = a[:, 0].reshape(N)
    adst = a[:, 1].reshape(N)
    u, sflat = _edge_kernel()(h, asrc, adst, cvec, src_pad, dst_pad)
    return u, sflat.reshape(2, N, 1)


def kernel(x, edge_index, W1, att_src1, att_dst1, b1, bn_gamma, bn_beta,
           W2, att_src2, att_dst2, b2):
    loop = jnp.arange(N, dtype=jnp.int32)
    padz = jnp.zeros((EPAD - REAL_E,), jnp.int32)
    src_pad = jnp.concatenate([edge_index[0], loop, padz]).reshape(-1, BLK)
    dst_pad = jnp.concatenate([edge_index[1], loop, padz]).reshape(-1, BLK)

    h1, a1, m1 = _project(x, W1, att_src1, att_dst1)
    u1, s1 = _gat_edges(h1, a1, m1, src_pad, dst_pad)

    gscaled = bn_gamma * (1.0 / jnp.sqrt(1.0 + 1e-5)).astype(jnp.float32)
    h2, a2, m2 = _mid(u1, s1, b1, gscaled, bn_beta, W2, att_src2, att_dst2)
    u2, s2 = _gat_edges(h2, a2, m2, src_pad, dst_pad)

    return _final(u2, s2, b2)
